# baseline (speedup 1.0000x reference)
import functools

import jax
import jax.numpy as jnp
from jax import lax
from jax.experimental import pallas as pl
from jax.experimental.pallas import tpu as pltpu


def _silu(x):
    return x * (1.0 / (1.0 + jnp.exp(-x)))


def _pq_body(nodes_ref, wa_ref, wb_ref, p_ref, q_ref):
    n = nodes_ref[...]
    p_ref[...] = jnp.dot(n, wa_ref[...], preferred_element_type=jnp.float32)
    q_ref[...] = jnp.dot(n, wb_ref[...], preferred_element_type=jnp.float32)


def _compute_pq(nodes, wa, wb, bn_blk=2000):
    n, d = nodes.shape
    grid = n // bn_blk
    return pl.pallas_call(
        _pq_body,
        grid=(grid,),
        in_specs=[
            pl.BlockSpec((bn_blk, d), lambda i: (i, 0)),
            pl.BlockSpec((d, d), lambda i: (0, 0)),
            pl.BlockSpec((d, d), lambda i: (0, 0)),
        ],
        out_specs=[
            pl.BlockSpec((bn_blk, d), lambda i: (i, 0)),
            pl.BlockSpec((bn_blk, d), lambda i: (i, 0)),
        ],
        out_shape=[
            jax.ShapeDtypeStruct((n, d), jnp.float32),
            jax.ShapeDtypeStruct((n, d), jnp.float32),
        ],
    )(nodes, wa, wb)


def _edge_body(gs_ref, gr_ref, ps_ref, pr_ref, ea_ref,
               wr_ref, wd_ref, b1_ref, w2_ref, b2_ref,
               wp1_ref, bp1_ref, wp2_ref,
               edges_ref, trans_ref):
    cd = ps_ref[...] - pr_ref[...]
    radial = jnp.sum(cd * cd, axis=1, keepdims=True)
    m = (gs_ref[...] + gr_ref[...]
         + radial * wr_ref[...]
         + jnp.dot(ea_ref[...], wd_ref[...], preferred_element_type=jnp.float32)
         + b1_ref[...])
    h = _silu(m)
    edges = _silu(jnp.dot(h, w2_ref[...], preferred_element_type=jnp.float32)
                  + b2_ref[...])
    edges_ref[...] = edges
    p = _silu(jnp.dot(edges, wp1_ref[...], preferred_element_type=jnp.float32)
              + bp1_ref[...])
    scale = jnp.dot(p, wp2_ref[...], preferred_element_type=jnp.float32)[:, 0:1]
    trans_ref[...] = jnp.clip(cd * scale, -100.0, 100.0)


def _edge_mlp(gs, gr, ps, pr, ea, wr, wd, b1, w2, b2, wp1, bp1, wp2p,
              be_blk=4000):
    e, d = gs.shape
    de = ea.shape[1]
    grid = e // be_blk
    full = lambda i: (0, 0)
    return pl.pallas_call(
        _edge_body,
        grid=(grid,),
        in_specs=[
            pl.BlockSpec((be_blk, d), lambda i: (i, 0)),
            pl.BlockSpec((be_blk, d), lambda i: (i, 0)),
            pl.BlockSpec((be_blk, 16), lambda i: (i, 0)),
            pl.BlockSpec((be_blk, 16), lambda i: (i, 0)),
            pl.BlockSpec((be_blk, de), lambda i: (i, 0)),
            pl.BlockSpec((1, d), full),
            pl.BlockSpec((de, d), full),
            pl.BlockSpec((1, d), full),
            pl.BlockSpec((d, d), full),
            pl.BlockSpec((1, d), full),
            pl.BlockSpec((d, d), full),
            pl.BlockSpec((1, d), full),
            pl.BlockSpec((d, 8), full),
        ],
        out_specs=[
            pl.BlockSpec((be_blk, d), lambda i: (i, 0)),
            pl.BlockSpec((be_blk, 16), lambda i: (i, 0)),
        ],
        out_shape=[
            jax.ShapeDtypeStruct((e, d), jnp.float32),
            jax.ShapeDtypeStruct((e, 16), jnp.float32),
        ],
    )(gs, gr, ps, pr, ea, wr, wd, b1, w2, b2, wp1, bp1, wp2p)


def _node_body(nodes_ref, agg_ref, pos_ref, pd_ref,
               wn1a_ref, wn1b_ref, bn1_ref, wn2_ref, bn2_ref,
               nn_ref, np_ref):
    nodes = nodes_ref[...]
    h2 = _silu(jnp.dot(nodes, wn1a_ref[...], preferred_element_type=jnp.float32)
               + jnp.dot(agg_ref[...], wn1b_ref[...], preferred_element_type=jnp.float32)
               + bn1_ref[...])
    nn_ref[...] = nodes + jnp.dot(h2, wn2_ref[...], preferred_element_type=jnp.float32) + bn2_ref[...]
    np_ref[...] = pos_ref[...] + pd_ref[...][:, 0:3]


def _node_update(nodes, agg, pos, pdel, wn1a, wn1b, bn1, wn2, bn2, bn_blk=2000):
    n, d = nodes.shape
    grid = n // bn_blk
    full = lambda i: (0, 0)
    return pl.pallas_call(
        _node_body,
        grid=(grid,),
        in_specs=[
            pl.BlockSpec((bn_blk, d), lambda i: (i, 0)),
            pl.BlockSpec((bn_blk, d), lambda i: (i, 0)),
            pl.BlockSpec((bn_blk, 3), lambda i: (i, 0)),
            pl.BlockSpec((bn_blk, 16), lambda i: (i, 0)),
            pl.BlockSpec((d, d), full),
            pl.BlockSpec((d, d), full),
            pl.BlockSpec((1, d), full),
            pl.BlockSpec((d, d), full),
            pl.BlockSpec((1, d), full),
        ],
        out_specs=[
            pl.BlockSpec((bn_blk, d), lambda i: (i, 0)),
            pl.BlockSpec((bn_blk, 3), lambda i: (i, 0)),
        ],
        out_shape=[
            jax.ShapeDtypeStruct((n, d), jnp.float32),
            jax.ShapeDtypeStruct((n, 3), jnp.float32),
        ],
    )(nodes, agg, pos, pdel, wn1a, wn1b, bn1, wn2, bn2)


def kernel(nodes, pos, edge_attr, We1, be1, We2, be2, Wn1, bn1, Wn2, bn2,
           Wp1, bp1, Wp2, senders, receivers):
    n, d = nodes.shape
    senders = senders.astype(jnp.int32)
    receivers = receivers.astype(jnp.int32)

    wa = We1[:d]
    wb = We1[d:2 * d]
    wr = We1[2 * d:2 * d + 1]
    wd = We1[2 * d + 1:]
    wp2p = jnp.pad(Wp2, ((0, 0), (0, 7)))
    pos16 = jnp.pad(pos, ((0, 0), (0, 13)))

    P, Q = _compute_pq(nodes, wa, wb)

    gs = jnp.take(P, senders, axis=0)
    gr = jnp.take(Q, receivers, axis=0)
    ps = jnp.take(pos16, senders, axis=0)
    pr = jnp.take(pos16, receivers, axis=0)

    edges, trans = _edge_mlp(
        gs, gr, ps, pr, edge_attr,
        wr, wd, be1.reshape(1, d), We2, be2.reshape(1, d),
        Wp1, bp1.reshape(1, d), wp2p)

    agg = jax.ops.segment_sum(edges, receivers, num_segments=n)
    pdel = jax.ops.segment_sum(trans, senders, num_segments=n)

    new_nodes, new_pos = _node_update(
        nodes, agg, pos, pdel,
        Wn1[:d], Wn1[d:], bn1.reshape(1, d), Wn2, bn2.reshape(1, d))
    return (new_nodes, new_pos)


# speedup vs baseline: 1.3287x; 1.3287x over previous
import functools

import jax
import jax.numpy as jnp
from jax import lax
from jax.experimental import pallas as pl
from jax.experimental.pallas import tpu as pltpu
from jax.experimental.pallas import tpu_sc as plsc

_NC = 2
_NS = 16
_NW = _NC * _NS


def _sc_gather(P, Q, pos4, s3, r3):
    n, d = P.shape
    nw, nch, ch = s3.shape
    e = nw * nch * ch
    epw = nch * ch
    ngrp = ch // 16
    mesh = plsc.VectorSubcoreMesh(core_axis_name="c", subcore_axis_name="s")

    def body(p_hbm, q_hbm, pos_hbm, s_hbm, r_hbm,
             gs_out, gr_out, rad_out,
             idx_s, idx_r,
             bufgs0, bufgs1, bufgr0, bufgr1,
             i4b0, i4b1, pgb0, pgb1, radb0, radb1,
             gsem0, gsem1, wsem0, wsem1):
        wid = lax.axis_index("c") * _NS + lax.axis_index("s")
        base = wid * epw
        pltpu.sync_copy(s_hbm.at[wid], idx_s)
        pltpu.sync_copy(r_hbm.at[wid], idx_r)

        slots = (
            ((p_hbm, idx_s, bufgs0, gs_out), (q_hbm, idx_r, bufgr0, gr_out)),
            ((p_hbm, idx_s, bufgs1, gs_out), (q_hbm, idx_r, bufgr1, gr_out)),
        )
        i4bufs = (i4b0, i4b1)
        pgbufs = (pgb0, pgb1)
        radbufs = (radb0, radb1)

        def g_start(j, slot, gsem):
            for tab, idx, buf, _ in slots[slot]:
                pltpu.make_async_copy(tab.at[idx.at[j]], buf, gsem).start()
            i4 = i4bufs[slot]
            for k in range(ngrp):
                sl16 = pl.ds(k * 16, 16)
                ids = idx_s[j, sl16] * 4
                idr = idx_r[j, sl16] * 4
                for c in range(3):
                    i4[pl.ds(c * ch + k * 16, 16)] = ids + c
                    i4[pl.ds((3 + c) * ch + k * 16, 16)] = idr + c
            for c in range(6):
                pltpu.make_async_copy(pos_hbm.at[i4.at[pl.ds(c * ch, ch)]],
                                      pgbufs[slot].at[pl.ds(c * ch, ch)],
                                      gsem).start()

        def g_wait_wb_start(j, slot, gsem, wsem):
            for tab, idx, buf, out in slots[slot]:
                pltpu.make_async_copy(tab.at[idx.at[j]], buf, gsem).wait()
                pltpu.make_async_copy(buf, out.at[pl.ds(base + j * ch, ch)],
                                      wsem).start()
            i4 = i4bufs[slot]
            pg = pgbufs[slot]
            for c in range(6):
                pltpu.make_async_copy(pos_hbm.at[i4.at[pl.ds(c * ch, ch)]],
                                      pg.at[pl.ds(c * ch, ch)], gsem).wait()
            rb = radbufs[slot]
            for k in range(ngrp):
                k16 = k * 16
                dx = pg[pl.ds(0 * ch + k16, 16)] - pg[pl.ds(3 * ch + k16, 16)]
                dy = pg[pl.ds(1 * ch + k16, 16)] - pg[pl.ds(4 * ch + k16, 16)]
                dz = pg[pl.ds(2 * ch + k16, 16)] - pg[pl.ds(5 * ch + k16, 16)]
                rb[pl.ds(k16, 16)] = dx * dx + dy * dy + dz * dz
            pltpu.make_async_copy(rb, rad_out.at[pl.ds(base + j * ch, ch)],
                                  wsem).start()

        def wb_wait(j, slot, wsem):
            for tab, idx, buf, out in slots[slot]:
                pltpu.make_async_copy(buf, out.at[pl.ds(base + j * ch, ch)],
                                      wsem).wait()
            pltpu.make_async_copy(radbufs[slot],
                                  rad_out.at[pl.ds(base + j * ch, ch)],
                                  wsem).wait()

        g_start(0, 0, gsem0)

        def loop(j, carry):
            even = (j % 2) == 0

            @pl.when(even)
            def _():
                @pl.when(j >= 2)
                def _():
                    wb_wait(j - 2, 0, wsem0)
                g_start(j, 0, gsem0)
                g_wait_wb_start(j - 1, 1, gsem1, wsem1)

            @pl.when(jnp.logical_not(even))
            def _():
                @pl.when(j >= 2)
                def _():
                    wb_wait(j - 2, 1, wsem1)
                g_start(j, 1, gsem1)
                g_wait_wb_start(j - 1, 0, gsem0, wsem0)

            return carry

        lax.fori_loop(1, nch, loop, 0, unroll=False)

        last = nch - 1
        if last % 2 == 0:
            sl, gsl, wsl, osl, owsl = 0, gsem0, wsem0, 1, wsem1
        else:
            sl, gsl, wsl, osl, owsl = 1, gsem1, wsem1, 0, wsem0
        g_wait_wb_start(last, sl, gsl, wsl)
        wb_wait(last - 1, osl, owsl)
        wb_wait(last, sl, wsl)

    kern = functools.partial(
        pl.kernel,
        out_type=[
            jax.ShapeDtypeStruct((e, d), jnp.float32),
            jax.ShapeDtypeStruct((e, d), jnp.float32),
            jax.ShapeDtypeStruct((e,), jnp.float32),
        ],
        mesh=mesh,
        scratch_types=[
            pltpu.VMEM((nch, ch), jnp.int32),
            pltpu.VMEM((nch, ch), jnp.int32),
            pltpu.VMEM((ch, d), jnp.float32),
            pltpu.VMEM((ch, d), jnp.float32),
            pltpu.VMEM((ch, d), jnp.float32),
            pltpu.VMEM((ch, d), jnp.float32),
            pltpu.VMEM((6 * ch,), jnp.int32),
            pltpu.VMEM((6 * ch,), jnp.int32),
            pltpu.VMEM((6 * ch,), jnp.float32),
            pltpu.VMEM((6 * ch,), jnp.float32),
            pltpu.VMEM((ch,), jnp.float32),
            pltpu.VMEM((ch,), jnp.float32),
            pltpu.SemaphoreType.DMA,
            pltpu.SemaphoreType.DMA,
            pltpu.SemaphoreType.DMA,
            pltpu.SemaphoreType.DMA,
        ],
    )(body)
    gs, gr, rad = kern(P, Q, pos4.reshape(-1), s3, r3)
    return gs, gr, rad


def _silu(x):
    return x * (1.0 / (1.0 + jnp.exp(-x)))


def _pq_body(nodes_ref, wa_ref, wb_ref, p_ref, q_ref):
    n = nodes_ref[...]
    p_ref[...] = jnp.dot(n, wa_ref[...], preferred_element_type=jnp.float32)
    q_ref[...] = jnp.dot(n, wb_ref[...], preferred_element_type=jnp.float32)


def _compute_pq(nodes, wa, wb, bn_blk=2000):
    n, d = nodes.shape
    grid = n // bn_blk
    return pl.pallas_call(
        _pq_body,
        grid=(grid,),
        in_specs=[
            pl.BlockSpec((bn_blk, d), lambda i: (i, 0)),
            pl.BlockSpec((d, d), lambda i: (0, 0)),
            pl.BlockSpec((d, d), lambda i: (0, 0)),
        ],
        out_specs=[
            pl.BlockSpec((bn_blk, d), lambda i: (i, 0)),
            pl.BlockSpec((bn_blk, d), lambda i: (i, 0)),
        ],
        out_shape=[
            jax.ShapeDtypeStruct((n, d), jnp.float32),
            jax.ShapeDtypeStruct((n, d), jnp.float32),
        ],
    )(nodes, wa, wb)


def _edge_body(gs_ref, gr_ref, rad_ref, ea_ref,
               wr_ref, wd_ref, b1_ref, w2_ref, b2_ref,
               wp1_ref, bp1_ref, wp2_ref,
               edges_ref, scale_ref):
    radial = rad_ref[...]
    m = (gs_ref[...] + gr_ref[...]
         + radial * wr_ref[...]
         + jnp.dot(ea_ref[...], wd_ref[...], preferred_element_type=jnp.float32)
         + b1_ref[...])
    h = _silu(m)
    edges = _silu(jnp.dot(h, w2_ref[...], preferred_element_type=jnp.float32)
                  + b2_ref[...])
    edges_ref[...] = edges
    p = _silu(jnp.dot(edges, wp1_ref[...], preferred_element_type=jnp.float32)
              + bp1_ref[...])
    scale_ref[...] = jnp.dot(p, wp2_ref[...], preferred_element_type=jnp.float32)[:, 0:1]


def _edge_mlp(gs, gr, rad, ea, wr, wd, b1, w2, b2, wp1, bp1, wp2p,
              be_blk=4000):
    e, d = gs.shape
    de = ea.shape[1]
    grid = e // be_blk
    full = lambda i: (0, 0)
    return pl.pallas_call(
        _edge_body,
        grid=(grid,),
        in_specs=[
            pl.BlockSpec((be_blk, d), lambda i: (i, 0)),
            pl.BlockSpec((be_blk, d), lambda i: (i, 0)),
            pl.BlockSpec((be_blk, 1), lambda i: (i, 0)),
            pl.BlockSpec((be_blk, de), lambda i: (i, 0)),
            pl.BlockSpec((1, d), full),
            pl.BlockSpec((de, d), full),
            pl.BlockSpec((1, d), full),
            pl.BlockSpec((d, d), full),
            pl.BlockSpec((1, d), full),
            pl.BlockSpec((d, d), full),
            pl.BlockSpec((1, d), full),
            pl.BlockSpec((d, 8), full),
        ],
        out_specs=[
            pl.BlockSpec((be_blk, d), lambda i: (i, 0)),
            pl.BlockSpec((be_blk, 1), lambda i: (i, 0)),
        ],
        out_shape=[
            jax.ShapeDtypeStruct((e, d), jnp.float32),
            jax.ShapeDtypeStruct((e, 1), jnp.float32),
        ],
    )(gs, gr, rad, ea, wr, wd, b1, w2, b2, wp1, bp1, wp2p)


def _node_body(nodes_ref, agg_ref, pos_ref, pd_ref,
               wn1a_ref, wn1b_ref, bn1_ref, wn2_ref, bn2_ref,
               nn_ref, np_ref):
    nodes = nodes_ref[...]
    h2 = _silu(jnp.dot(nodes, wn1a_ref[...], preferred_element_type=jnp.float32)
               + jnp.dot(agg_ref[...], wn1b_ref[...], preferred_element_type=jnp.float32)
               + bn1_ref[...])
    nn_ref[...] = nodes + jnp.dot(h2, wn2_ref[...], preferred_element_type=jnp.float32) + bn2_ref[...]
    np_ref[...] = pos_ref[...] + pd_ref[...][:, 0:3]


def _node_update(nodes, agg, pos, pdel, wn1a, wn1b, bn1, wn2, bn2, bn_blk=2000):
    n, d = nodes.shape
    grid = n // bn_blk
    full = lambda i: (0, 0)
    return pl.pallas_call(
        _node_body,
        grid=(grid,),
        in_specs=[
            pl.BlockSpec((bn_blk, d), lambda i: (i, 0)),
            pl.BlockSpec((bn_blk, d), lambda i: (i, 0)),
            pl.BlockSpec((bn_blk, 3), lambda i: (i, 0)),
            pl.BlockSpec((bn_blk, 4), lambda i: (i, 0)),
            pl.BlockSpec((d, d), full),
            pl.BlockSpec((d, d), full),
            pl.BlockSpec((1, d), full),
            pl.BlockSpec((d, d), full),
            pl.BlockSpec((1, d), full),
        ],
        out_specs=[
            pl.BlockSpec((bn_blk, d), lambda i: (i, 0)),
            pl.BlockSpec((bn_blk, 3), lambda i: (i, 0)),
        ],
        out_shape=[
            jax.ShapeDtypeStruct((n, d), jnp.float32),
            jax.ShapeDtypeStruct((n, 3), jnp.float32),
        ],
    )(nodes, agg, pos, pdel, wn1a, wn1b, bn1, wn2, bn2)


def kernel(nodes, pos, edge_attr, We1, be1, We2, be2, Wn1, bn1, Wn2, bn2,
           Wp1, bp1, Wp2, senders, receivers):
    n, d = nodes.shape
    senders = senders.astype(jnp.int32)
    receivers = receivers.astype(jnp.int32)

    wa = We1[:d]
    wb = We1[d:2 * d]
    wr = We1[2 * d:2 * d + 1]
    wd = We1[2 * d + 1:]
    wp2p = jnp.pad(Wp2, ((0, 0), (0, 7)))
    pos4 = jnp.pad(pos, ((0, 0), (0, 1)))

    P, Q = _compute_pq(nodes, wa, wb)

    e = senders.shape[0]
    epw = e // _NW
    ch = 80
    nch = epw // ch
    s3 = senders.reshape(_NW, nch, ch)
    r3 = receivers.reshape(_NW, nch, ch)
    gs, gr, rad = _sc_gather(P, Q, pos4, s3, r3)

    edges, scale = _edge_mlp(
        gs, gr, rad.reshape(e, 1), edge_attr,
        wr, wd, be1.reshape(1, d), We2, be2.reshape(1, d),
        Wp1, bp1.reshape(1, d), wp2p)

    agg = jax.ops.segment_sum(edges, receivers, num_segments=n)
    cdfull = pos4[senders] - pos4[receivers]
    trans = jnp.clip(cdfull * scale, -100.0, 100.0)
    pdel = jax.ops.segment_sum(trans, senders, num_segments=n)

    new_nodes, new_pos = _node_update(
        nodes, agg, pos, pdel,
        Wn1[:d], Wn1[d:], bn1.reshape(1, d), Wn2, bn2.reshape(1, d))
    return (new_nodes, new_pos)
